# baseline (speedup 1.0000x reference)
import functools

import jax
import jax.numpy as jnp
from jax import lax
from jax.experimental import pallas as pl
from jax.experimental.pallas import tpu as pltpu
from jax.experimental.pallas import tpu_sc as plsc

EMBED_DIM = 32
LANES = 16
NUM_CORES = 2
NUM_SUBCORES = 16
NUM_WORKERS = NUM_CORES * NUM_SUBCORES
CHUNK = 128
SUB = EMBED_DIM // 8
TILE_ELEMS = CHUNK * EMBED_DIM
PAIR = 2 * CHUNK
PAIR_ELEMS = 2 * TILE_ELEMS


def _make_relayout(vocab: int):
    n_full = vocab // CHUNK
    rem = vocab - n_full * CHUNK
    per_w = n_full // NUM_WORKERS
    per_w -= per_w % 4
    tiles_main = per_w * NUM_WORKERS
    tail_full = n_full - tiles_main
    assert tail_full % 2 == 0 and tail_full // 2 <= NUM_WORKERS
    pairs_per_w = per_w // 2
    n_half = pairs_per_w // 2
    mesh = plsc.VectorSubcoreMesh(core_axis_name="c", subcore_axis_name="s")

    @functools.partial(
        pl.kernel,
        out_type=jax.ShapeDtypeStruct((vocab * EMBED_DIM,), jnp.float32),
        mesh=mesh,
        scratch_types=[
            pltpu.VMEM((EMBED_DIM, PAIR), jnp.float32),
            pltpu.VMEM((EMBED_DIM, PAIR), jnp.float32),
            pltpu.VMEM((PAIR_ELEMS,), jnp.float32),
            pltpu.VMEM((PAIR_ELEMS,), jnp.float32),
            pltpu.SemaphoreType.DMA,
            pltpu.SemaphoreType.DMA,
            pltpu.SemaphoreType.DMA,
            pltpu.SemaphoreType.DMA,
        ],
        compiler_params=pltpu.CompilerParams(
            use_tc_tiling_on_sc=True, needs_layout_passes=False
        ),
    )
    def relayout(
        tab_t_hbm, tail_hbm, dense_hbm, t0v, t1v, d0v, d1v, sg0, sg1, ss0, ss1
    ):
        tvs, dvs = (t0v, t1v), (d0v, d1v)
        sgs, sss = (sg0, sg1), (ss0, ss1)
        wid = lax.axis_index("s") * NUM_CORES + lax.axis_index("c")
        base = wid * pairs_per_w
        iota = lax.iota(jnp.int32, LANES)

        xks = [iota ^ k for k in range(LANES)]
        xks32 = [xk * EMBED_DIM + iota for xk in xks]

        def transpose_pair(src, dst):
            n_blocks = (EMBED_DIM // LANES) * (PAIR // LANES)

            @plsc.parallel_loop(0, n_blocks, unroll=4)
            def _(blk):
                j0 = (blk & 1) * LANES
                c0 = (blk >> 1) * LANES
                rows = jnp.full((LANES,), j0, jnp.int32) + iota
                base_c = jnp.full((LANES,), c0, jnp.int32)
                base_d = jnp.full((LANES,), c0 * EMBED_DIM + j0, jnp.int32)
                for k in range(LANES):
                    v = plsc.load_gather(src, [rows, base_c + xks[k]])
                    plsc.store_scatter(dst, [base_d + xks32[k]], v)

        pltpu.async_copy(tab_t_hbm.at[:, pl.ds(base * PAIR, PAIR)], t0v, sg0)
        pltpu.async_copy(
            tab_t_hbm.at[:, pl.ds((base + 1) * PAIR, PAIR)], t1v, sg1
        )

        def iter_body(i, carry):
            for b in range(2):
                p = base + 2 * i + b
                pltpu.make_async_copy(
                    tab_t_hbm.at[:, pl.ds(0, PAIR)], tvs[b], sgs[b]
                ).wait()
                @pl.when(i > 0)
                def _():
                    pltpu.make_async_copy(
                        dvs[b], dense_hbm.at[pl.ds(0, PAIR_ELEMS)], sss[b]
                    ).wait()

                transpose_pair(tvs[b], dvs[b])

                @pl.when(i < n_half - 1)
                def _():
                    pltpu.async_copy(
                        tab_t_hbm.at[:, pl.ds((p + 2) * PAIR, PAIR)],
                        tvs[b],
                        sgs[b],
                    )

                pltpu.async_copy(
                    dvs[b], dense_hbm.at[pl.ds(p * PAIR_ELEMS, PAIR_ELEMS)], sss[b]
                )
            return carry

        lax.fori_loop(0, n_half, iter_body, 0)
        for b in range(2):
            pltpu.make_async_copy(
                dvs[b], dense_hbm.at[pl.ds(0, PAIR_ELEMS)], sss[b]
            ).wait()

        @pl.when(wid < tail_full // 2)
        def _():
            p = (tiles_main // 2) + wid
            pltpu.sync_copy(tab_t_hbm.at[:, pl.ds(p * PAIR, PAIR)], t0v)
            transpose_pair(t0v, d0v)
            pltpu.sync_copy(d0v, dense_hbm.at[pl.ds(p * PAIR_ELEMS, PAIR_ELEMS)])

        if rem:
            @pl.when(wid == tail_full // 2)
            def _():
                start = n_full * TILE_ELEMS
                n = rem * EMBED_DIM
                pltpu.sync_copy(tail_hbm, d1v.at[pl.ds(0, n)])
                pltpu.sync_copy(
                    d1v.at[pl.ds(0, n)], dense_hbm.at[pl.ds(start, n)]
                )

    return relayout


def _make_lookup(fields: int, n_tiles: int):
    n_chunks = fields * n_tiles
    assert n_tiles % 2 == 0 and n_chunks % (NUM_WORKERS * 4) == 0
    pairs_per_w = n_chunks // (2 * NUM_WORKERS)
    NBUF = 4
    assert pairs_per_w % NBUF == 0
    n_rounds = pairs_per_w // NBUF
    n_tpairs = n_tiles // 2
    out_elems = n_chunks * TILE_ELEMS
    sub_sz = 8 * CHUNK
    mesh = plsc.VectorSubcoreMesh(core_axis_name="c", subcore_axis_name="s")

    @functools.partial(
        pl.kernel,
        out_type=jax.ShapeDtypeStruct((out_elems,), jnp.float32),
        mesh=mesh,
        scratch_types=[
            pltpu.VMEM((2 * pairs_per_w, CHUNK), jnp.int32),
            *([pltpu.VMEM((PAIR, EMBED_DIM), jnp.float32)] * 4),
            *([pltpu.VMEM((PAIR_ELEMS,), jnp.float32)] * 4),
            *([pltpu.SemaphoreType.DMA] * 8),
        ],
        compiler_params=pltpu.CompilerParams(
            use_tc_tiling_on_sc=False, needs_layout_passes=False
        ),
    )
    def lookup(
        table_hbm, idx_hbm, out_hbm,
        idx_v, in0, in1, in2, in3, tr0, tr1, tr2, tr3,
        sg0, sg1, sg2, sg3, ss0, ss1, ss2, ss3,
    ):
        ins, trs = (in0, in1, in2, in3), (tr0, tr1, tr2, tr3)
        sgs, sss = (sg0, sg1, sg2, sg3), (ss0, ss1, ss2, ss3)
        wid = lax.axis_index("s") * NUM_CORES + lax.axis_index("c")
        base = wid * pairs_per_w
        pltpu.sync_copy(
            idx_hbm.at[pl.ds(base * 2, 2 * pairs_per_w), :], idx_v
        )
        iota = lax.iota(jnp.int32, LANES)
        n_blocks_tr = (EMBED_DIM // LANES) * (PAIR // LANES)
        xks = [iota ^ k for k in range(LANES)]
        xks128 = [xk * CHUNK for xk in xks]

        def gather_pair(p, b):
            pltpu.async_copy(
                table_hbm.at[idx_v.at[2 * p]], ins[b].at[pl.ds(0, CHUNK), :],
                sgs[b],
            )
            pltpu.async_copy(
                table_hbm.at[idx_v.at[2 * p + 1]],
                ins[b].at[pl.ds(CHUNK, CHUNK), :],
                sgs[b],
            )

        for b in range(NBUF):
            gather_pair(b, b)

        def iter_body(i, carry):
            for b in range(NBUF):
                j = NBUF * i + b
                pr = base + j
                f = pr // n_tpairs
                tp = lax.rem(pr, n_tpairs)
                pltpu.make_async_copy(
                    table_hbm.at[pl.ds(0, PAIR), :], ins[b], sgs[b]
                ).wait()
                @pl.when(i > 0)
                def _():
                    pltpu.make_async_copy(
                        trs[b], out_hbm.at[pl.ds(0, PAIR_ELEMS)], sss[b]
                    ).wait()

                @plsc.parallel_loop(0, n_blocks_tr, unroll=4)
                def _(blk):
                    j0 = (blk & 1) * LANES
                    c0 = (blk >> 1) * LANES
                    rows = jnp.full((LANES,), c0, jnp.int32) + iota
                    dbase = (
                        (c0 >> 7) * (SUB * sub_sz)
                        + (c0 & (CHUNK - 1))
                        + j0 * CHUNK
                    )
                    base_d = jnp.full((LANES,), dbase, jnp.int32) + iota
                    base_j = jnp.full((LANES,), j0, jnp.int32)
                    for k in range(LANES):
                        v = jnp.abs(
                            plsc.load_gather(ins[b], [rows, base_j + xks[k]])
                        )
                        plsc.store_scatter(trs[b], [base_d + xks128[k]], v)

                @pl.when(i < n_rounds - 1)
                def _():
                    gather_pair(j + NBUF, b)

                for a in range(SUB):
                    for h in range(2):
                        pltpu.async_copy(
                            trs[b].at[
                                pl.ds((h * SUB + a) * sub_sz, sub_sz)
                            ],
                            out_hbm.at[
                                pl.ds(
                                    (2 * ((f * SUB + a) * n_tpairs + tp) + h)
                                    * sub_sz,
                                    sub_sz,
                                )
                            ],
                            sss[b],
                        )
            return carry

        lax.fori_loop(0, n_rounds, iter_body, 0)
        for b in range(NBUF):
            pltpu.make_async_copy(
                trs[b], out_hbm.at[pl.ds(0, PAIR_ELEMS)], sss[b]
            ).wait()

    return lookup


def kernel(inputs, table):
    batch, fields = inputs.shape
    vocab = table.shape[0]
    n_tiles = batch // CHUNK
    rem = vocab % CHUNK
    tail1d = table[vocab - rem :].reshape(-1) if rem else jnp.zeros(
        (EMBED_DIM,), jnp.float32
    )
    dense1d = _make_relayout(vocab)(table.T, tail1d)
    dense2d = dense1d.reshape(vocab, EMBED_DIM)
    idx2d = inputs.T.reshape(fields * n_tiles, CHUNK).astype(jnp.int32)
    out1d = _make_lookup(fields, n_tiles)(dense2d, idx2d)
    out5 = out1d.reshape(fields, SUB, n_tiles, 8, CHUNK)
    return out5.transpose(2, 4, 0, 1, 3).reshape(batch, fields, EMBED_DIM)


# speedup vs baseline: 1.0085x; 1.0085x over previous
import functools

import jax
import jax.numpy as jnp
from jax import lax
from jax.experimental import pallas as pl
from jax.experimental.pallas import tpu as pltpu
from jax.experimental.pallas import tpu_sc as plsc

EMBED_DIM = 32
LANES = 16
NUM_CORES = 2
NUM_SUBCORES = 16
NUM_WORKERS = NUM_CORES * NUM_SUBCORES
CHUNK = 128
SUB = EMBED_DIM // 8
TILE_ELEMS = CHUNK * EMBED_DIM
PAIR = 2 * CHUNK
PAIR_ELEMS = 2 * TILE_ELEMS


def _make_relayout(vocab: int):
    n_full = vocab // CHUNK
    rem = vocab - n_full * CHUNK
    per_w = n_full // NUM_WORKERS
    per_w -= per_w % 4
    tiles_main = per_w * NUM_WORKERS
    tail_full = n_full - tiles_main
    assert tail_full % 2 == 0 and tail_full // 2 <= NUM_WORKERS
    pairs_per_w = per_w // 2
    n_half = pairs_per_w // 2
    mesh = plsc.VectorSubcoreMesh(core_axis_name="c", subcore_axis_name="s")

    @functools.partial(
        pl.kernel,
        out_type=jax.ShapeDtypeStruct((vocab * EMBED_DIM,), jnp.float32),
        mesh=mesh,
        scratch_types=[
            pltpu.VMEM((EMBED_DIM, PAIR), jnp.float32),
            pltpu.VMEM((EMBED_DIM, PAIR), jnp.float32),
            pltpu.VMEM((PAIR_ELEMS,), jnp.float32),
            pltpu.VMEM((PAIR_ELEMS,), jnp.float32),
            pltpu.SemaphoreType.DMA,
            pltpu.SemaphoreType.DMA,
            pltpu.SemaphoreType.DMA,
            pltpu.SemaphoreType.DMA,
        ],
        compiler_params=pltpu.CompilerParams(
            use_tc_tiling_on_sc=True, needs_layout_passes=False
        ),
    )
    def relayout(
        tab_t_hbm, tail_hbm, dense_hbm, t0v, t1v, d0v, d1v, sg0, sg1, ss0, ss1
    ):
        tvs, dvs = (t0v, t1v), (d0v, d1v)
        sgs, sss = (sg0, sg1), (ss0, ss1)
        wid = lax.axis_index("s") * NUM_CORES + lax.axis_index("c")
        base = wid * pairs_per_w
        iota = lax.iota(jnp.int32, LANES)

        xks = [iota ^ k for k in range(LANES)]
        xks32 = [xk * EMBED_DIM + iota for xk in xks]

        def transpose_pair(src, dst):
            n_blocks = (EMBED_DIM // LANES) * (PAIR // LANES)

            @plsc.parallel_loop(0, n_blocks, unroll=4)
            def _(blk):
                j0 = (blk & 1) * LANES
                c0 = (blk >> 1) * LANES
                rows = jnp.full((LANES,), j0, jnp.int32) + iota
                base_c = jnp.full((LANES,), c0, jnp.int32)
                base_d = jnp.full((LANES,), c0 * EMBED_DIM + j0, jnp.int32)
                for k in range(LANES):
                    v = plsc.load_gather(src, [rows, base_c + xks[k]])
                    plsc.store_scatter(dst, [base_d + xks32[k]], v)

        pltpu.async_copy(tab_t_hbm.at[:, pl.ds(base * PAIR, PAIR)], t0v, sg0)
        pltpu.async_copy(
            tab_t_hbm.at[:, pl.ds((base + 1) * PAIR, PAIR)], t1v, sg1
        )

        def iter_body(i, carry):
            for b in range(2):
                p = base + 2 * i + b
                pltpu.make_async_copy(
                    tab_t_hbm.at[:, pl.ds(0, PAIR)], tvs[b], sgs[b]
                ).wait()
                @pl.when(i > 0)
                def _():
                    pltpu.make_async_copy(
                        dvs[b], dense_hbm.at[pl.ds(0, PAIR_ELEMS)], sss[b]
                    ).wait()

                transpose_pair(tvs[b], dvs[b])

                @pl.when(i < n_half - 1)
                def _():
                    pltpu.async_copy(
                        tab_t_hbm.at[:, pl.ds((p + 2) * PAIR, PAIR)],
                        tvs[b],
                        sgs[b],
                    )

                pltpu.async_copy(
                    dvs[b], dense_hbm.at[pl.ds(p * PAIR_ELEMS, PAIR_ELEMS)], sss[b]
                )
            return carry

        lax.fori_loop(0, n_half, iter_body, 0)
        for b in range(2):
            pltpu.make_async_copy(
                dvs[b], dense_hbm.at[pl.ds(0, PAIR_ELEMS)], sss[b]
            ).wait()

        @pl.when(wid < tail_full // 2)
        def _():
            p = (tiles_main // 2) + wid
            pltpu.sync_copy(tab_t_hbm.at[:, pl.ds(p * PAIR, PAIR)], t0v)
            transpose_pair(t0v, d0v)
            pltpu.sync_copy(d0v, dense_hbm.at[pl.ds(p * PAIR_ELEMS, PAIR_ELEMS)])

        if rem:
            @pl.when(wid == tail_full // 2)
            def _():
                start = n_full * TILE_ELEMS
                n = rem * EMBED_DIM
                pltpu.sync_copy(tail_hbm, d1v.at[pl.ds(0, n)])
                pltpu.sync_copy(
                    d1v.at[pl.ds(0, n)], dense_hbm.at[pl.ds(start, n)]
                )

    return relayout


def _make_lookup(fields: int, n_tiles: int):
    n_chunks = fields * n_tiles
    assert n_tiles % 2 == 0 and n_chunks % (NUM_WORKERS * 4) == 0
    pairs_per_w = n_chunks // (2 * NUM_WORKERS)
    NBUF = 2
    n_rounds = pairs_per_w // NBUF
    n_tpairs = n_tiles // 2
    out_elems = n_chunks * TILE_ELEMS
    sub_sz = 8 * CHUNK
    mesh = plsc.VectorSubcoreMesh(core_axis_name="c", subcore_axis_name="s")

    @functools.partial(
        pl.kernel,
        out_type=jax.ShapeDtypeStruct((out_elems,), jnp.float32),
        mesh=mesh,
        scratch_types=[
            pltpu.VMEM((2 * pairs_per_w, CHUNK), jnp.int32),
            *([pltpu.VMEM((PAIR, EMBED_DIM), jnp.float32)] * 2),
            *([pltpu.VMEM((PAIR_ELEMS,), jnp.float32)] * 2),
            *([pltpu.SemaphoreType.DMA] * 4),
        ],
        compiler_params=pltpu.CompilerParams(
            use_tc_tiling_on_sc=False, needs_layout_passes=False
        ),
    )
    def lookup(
        table_hbm, idx_hbm, out_hbm,
        idx_v, in0, in1, tr0, tr1, sg0, sg1, ss0, ss1,
    ):
        ins, trs = (in0, in1), (tr0, tr1)
        sgs, sss = (sg0, sg1), (ss0, ss1)
        wid = lax.axis_index("s") * NUM_CORES + lax.axis_index("c")
        base = wid * pairs_per_w
        pltpu.sync_copy(
            idx_hbm.at[pl.ds(base * 2, 2 * pairs_per_w), :], idx_v
        )
        iota = lax.iota(jnp.int32, LANES)
        n_blocks_tr = (EMBED_DIM // LANES) * (PAIR // LANES)
        xks = [iota ^ k for k in range(LANES)]
        xks128 = [xk * CHUNK for xk in xks]

        def gather_pair(p, b):
            pltpu.async_copy(
                table_hbm.at[idx_v.at[2 * p]], ins[b].at[pl.ds(0, CHUNK), :],
                sgs[b],
            )
            pltpu.async_copy(
                table_hbm.at[idx_v.at[2 * p + 1]],
                ins[b].at[pl.ds(CHUNK, CHUNK), :],
                sgs[b],
            )

        for b in range(NBUF):
            gather_pair(b, b)

        def iter_body(i, carry):
            for b in range(NBUF):
                j = NBUF * i + b
                pr = base + j
                f = pr // n_tpairs
                tp = lax.rem(pr, n_tpairs)
                pltpu.make_async_copy(
                    table_hbm.at[pl.ds(0, PAIR), :], ins[b], sgs[b]
                ).wait()
                @pl.when(i > 0)
                def _():
                    pltpu.make_async_copy(
                        trs[b], out_hbm.at[pl.ds(0, PAIR_ELEMS)], sss[b]
                    ).wait()

                @plsc.parallel_loop(0, n_blocks_tr, unroll=4)
                def _(blk):
                    j0 = (blk & 1) * LANES
                    c0 = (blk >> 1) * LANES
                    rows = jnp.full((LANES,), c0, jnp.int32) + iota
                    dbase = (
                        (c0 >> 7) * (SUB * sub_sz)
                        + (c0 & (CHUNK - 1))
                        + j0 * CHUNK
                    )
                    base_d = jnp.full((LANES,), dbase, jnp.int32) + iota
                    base_j = jnp.full((LANES,), j0, jnp.int32)
                    for k in range(LANES):
                        v = jnp.abs(
                            plsc.load_gather(ins[b], [rows, base_j + xks[k]])
                        )
                        plsc.store_scatter(trs[b], [base_d + xks128[k]], v)

                @pl.when(i < n_rounds - 1)
                def _():
                    gather_pair(j + NBUF, b)

                for a in range(SUB):
                    for h in range(2):
                        pltpu.async_copy(
                            trs[b].at[
                                pl.ds((h * SUB + a) * sub_sz, sub_sz)
                            ],
                            out_hbm.at[
                                pl.ds(
                                    (2 * ((f * SUB + a) * n_tpairs + tp) + h)
                                    * sub_sz,
                                    sub_sz,
                                )
                            ],
                            sss[b],
                        )
            return carry

        lax.fori_loop(0, n_rounds, iter_body, 0)
        for b in range(NBUF):
            pltpu.make_async_copy(
                trs[b], out_hbm.at[pl.ds(0, PAIR_ELEMS)], sss[b]
            ).wait()

    return lookup


def kernel(inputs, table):
    batch, fields = inputs.shape
    vocab = table.shape[0]
    n_tiles = batch // CHUNK
    rem = vocab % CHUNK
    tail1d = table[vocab - rem :].reshape(-1) if rem else jnp.zeros(
        (EMBED_DIM,), jnp.float32
    )
    dense1d = _make_relayout(vocab)(table.T, tail1d)
    dense2d = dense1d.reshape(vocab, EMBED_DIM)
    idx2d = inputs.T.reshape(fields * n_tiles, CHUNK).astype(jnp.int32)
    out1d = _make_lookup(fields, n_tiles)(dense2d, idx2d)
    out5 = out1d.reshape(fields, SUB, n_tiles, 8, CHUNK)
    return out5.transpose(2, 4, 0, 1, 3).reshape(batch, fields, EMBED_DIM)
